# baseline (speedup 1.0000x reference)
import functools

import jax
import jax.numpy as jnp
import numpy as np
from jax import lax
from jax.experimental import pallas as pl
from jax.experimental.pallas import tpu as pltpu
from jax.experimental.pallas import tpu_sc as plsc

N_COARSE = 4096
N_FINE = 16384
D_IN = 256
D_SKIP = 128
D_HID = 256
D_OUT = 256
K = 3

BM = 1024
BMC = 512


def _topk_body(py_ref, px_ref, i_ref, w_ref):
    d = None
    for c in range(K):
        diff = py_ref[:, c:c + 1] - px_ref[c:c + 1, :]
        sq = diff * diff
        d = sq if d is None else d + sq
    n = d.shape[1]
    idxrow = lax.broadcasted_iota(jnp.int32, d.shape, 1).astype(jnp.float32)
    big_f = jnp.float32(n)
    inf = jnp.float32(np.inf)

    mins, idxs = [], []
    for _ in range(K):
        mk = jnp.min(d, axis=1, keepdims=True)
        cand = jnp.where(d == mk, idxrow, big_f)
        ik = jnp.min(cand, axis=1, keepdims=True)
        d = jnp.where(cand == ik, inf, d)
        mins.append(mk)
        idxs.append(ik)

    ws = [1.0 / jnp.maximum(mk, 1e-16) for mk in mins]
    den = ws[0] + ws[1] + ws[2]
    i_ref[...] = jnp.concatenate(
        [ik.astype(jnp.int32) for ik in idxs], axis=1)
    w_ref[...] = jnp.concatenate([wk / den for wk in ws], axis=1)


def _topk_call(py, px):
    m = py.shape[0]
    grid = (m // BM,)
    tri = pl.BlockSpec((BM, 3), lambda i: (i, 0))
    return pl.pallas_call(
        _topk_body,
        grid=grid,
        in_specs=[
            pl.BlockSpec((BM, 3), lambda i: (i, 0)),
            pl.BlockSpec((3, N_COARSE), lambda i: (0, 0)),
        ],
        out_specs=[tri, tri],
        out_shape=[jax.ShapeDtypeStruct((m, 3), jnp.int32),
                   jax.ShapeDtypeStruct((m, 3), jnp.float32)],
    )(py, px)


_NC = 2
_NS = 16
_NW = _NC * _NS
_GATHER_B = K * N_FINE
_B_PER_W = _GATHER_B // _NW
_CHUNK = 192
_N_CHUNKS = _B_PER_W // _CHUNK


def _sc_gather_body(table_hbm, idx_hbm, out_hbm,
                    idx0, idx1, rows0, rows1, sem0, sem1):
    wid = lax.axis_index("s") * _NC + lax.axis_index("c")
    base = wid * _B_PER_W
    idx_v = (idx0, idx1)
    rows_v = (rows0, rows1)
    sems = (sem0, sem1)
    pltpu.sync_copy(idx_hbm.at[pl.ds(base, _CHUNK)], idx0)
    cps = {0: pltpu.async_copy(table_hbm.at[idx0], rows0, sem0)}
    for ci in range(_N_CHUNKS):
        cur, nxt = ci % 2, (ci + 1) % 2
        if ci + 1 < _N_CHUNKS:
            off = base + (ci + 1) * _CHUNK
            pltpu.sync_copy(idx_hbm.at[pl.ds(off, _CHUNK)], idx_v[nxt])
            cps[nxt] = pltpu.async_copy(
                table_hbm.at[idx_v[nxt]], rows_v[nxt], sems[nxt])
        cps[cur].wait()
        pltpu.sync_copy(rows_v[cur], out_hbm.at[pl.ds(base + ci * _CHUNK, _CHUNK)])


@functools.cache
def _sc_gather():
    return functools.partial(
        pl.kernel,
        mesh=plsc.VectorSubcoreMesh(core_axis_name="c", subcore_axis_name="s"),
        out_type=jax.ShapeDtypeStruct((_GATHER_B, D_IN), jnp.float32),
        scratch_types=[
            pltpu.VMEM((_CHUNK,), jnp.int32),
            pltpu.VMEM((_CHUNK,), jnp.int32),
            pltpu.VMEM((_CHUNK, D_IN), jnp.float32),
            pltpu.VMEM((_CHUNK, D_IN), jnp.float32),
            pltpu.SemaphoreType.DMA,
            pltpu.SemaphoreType.DMA,
        ],
    )(_sc_gather_body)


def _mlp_body(g_ref, w_ref, xs_ref,
              w1a_ref, w1b_ref, b1_ref, w2m_ref, b2_ref, o_ref):
    w = w_ref[...]
    xi = (w[:, 0:1] * g_ref[:, 0, :]
          + w[:, 1:2] * g_ref[:, 1, :]
          + w[:, 2:3] * g_ref[:, 2, :])
    h = jnp.dot(xi, w1a_ref[...], preferred_element_type=jnp.float32)
    h = h + jnp.dot(xs_ref[...], w1b_ref[...],
                    preferred_element_type=jnp.float32)
    h = jnp.maximum(h + b1_ref[...][None, :], 0.0)
    o = jnp.dot(h, w2m_ref[...], preferred_element_type=jnp.float32)
    o_ref[...] = jnp.maximum(o + b2_ref[...][None, :], 0.0)


def _mlp_call(gathered3, w_all, x_skip, W1, b1, W2, b2):
    m = x_skip.shape[0]
    grid = (m // BMC,)
    return pl.pallas_call(
        _mlp_body,
        grid=grid,
        in_specs=[
            pl.BlockSpec((BMC, 3, D_IN), lambda i: (i, 0, 0)),
            pl.BlockSpec((BMC, 3), lambda i: (i, 0)),
            pl.BlockSpec((BMC, D_SKIP), lambda i: (i, 0)),
            pl.BlockSpec((D_IN, D_HID), lambda i: (0, 0)),
            pl.BlockSpec((D_SKIP, D_HID), lambda i: (2, 0)),
            pl.BlockSpec((D_HID,), lambda i: (0,)),
            pl.BlockSpec((D_HID, D_OUT), lambda i: (0, 0)),
            pl.BlockSpec((D_OUT,), lambda i: (0,)),
        ],
        out_specs=pl.BlockSpec((BMC, D_OUT), lambda i: (i, 0)),
        out_shape=jax.ShapeDtypeStruct((m, D_OUT), jnp.float32),
    )(gathered3, w_all, x_skip, W1, W1, b1, W2, b2)


def kernel(x, pos, batch, x_skip, pos_skip, batch_skip, W1, b1, W2, b2):
    m = pos_skip.shape[0]
    idx_all, w_all = _topk_call(pos_skip, pos.T)

    gathered = _sc_gather()(x, idx_all.reshape(-1))
    h = _mlp_call(gathered.reshape(m, K, D_IN), w_all, x_skip, W1, b1, W2, b2)
    return (h, pos_skip, batch_skip)


# speedup vs baseline: 1.2449x; 1.2449x over previous
import functools

import jax
import jax.numpy as jnp
import numpy as np
from jax import lax
from jax.experimental import pallas as pl
from jax.experimental.pallas import tpu as pltpu
from jax.experimental.pallas import tpu_sc as plsc

N_COARSE = 4096
N_FINE = 16384
D_IN = 256
D_SKIP = 128
D_HID = 256
D_OUT = 256
K = 3

BM = 1024
BMC = 512


def _topk_body(py_ref, px_ref, i_ref, w_ref):
    d = None
    for c in range(K):
        diff = py_ref[:, c:c + 1] - px_ref[c:c + 1, :]
        sq = diff * diff
        d = sq if d is None else d + sq
    n = d.shape[1]
    idxrow = lax.broadcasted_iota(jnp.int32, d.shape, 1).astype(jnp.float32)
    big_f = jnp.float32(n)
    inf = jnp.float32(np.inf)

    mins, idxs = [], []
    for _ in range(K):
        mk = jnp.min(d, axis=1, keepdims=True)
        cand = jnp.where(d == mk, idxrow, big_f)
        ik = jnp.min(cand, axis=1, keepdims=True)
        d = jnp.where(cand == ik, inf, d)
        mins.append(mk)
        idxs.append(ik)

    ws = [1.0 / jnp.maximum(mk, 1e-16) for mk in mins]
    den = ws[0] + ws[1] + ws[2]
    i_ref[...] = jnp.concatenate(
        [ik.astype(jnp.int32) for ik in idxs], axis=1)
    w_ref[...] = jnp.concatenate([wk / den for wk in ws], axis=1)


def _topk_call(py, px):
    m = py.shape[0]
    grid = (m // BM,)
    tri = pl.BlockSpec((BM, 3), lambda i: (i, 0))
    return pl.pallas_call(
        _topk_body,
        grid=grid,
        in_specs=[
            pl.BlockSpec((BM, 3), lambda i: (i, 0)),
            pl.BlockSpec((3, N_COARSE), lambda i: (0, 0)),
        ],
        out_specs=[tri, tri],
        out_shape=[jax.ShapeDtypeStruct((m, 3), jnp.int32),
                   jax.ShapeDtypeStruct((m, 3), jnp.float32)],
    )(py, px)


_NC = 2
_NS = 16
_NW = _NC * _NS
_GATHER_B = K * N_FINE
_B_PER_W = _GATHER_B // _NW
_CHUNK = 192
_N_CHUNKS = _B_PER_W // _CHUNK


def _sc_gather_body(table_hbm, idx_hbm, out_hbm,
                    idx0, idx1, rows0, rows1, sem0, sem1):
    wid = lax.axis_index("s") * _NC + lax.axis_index("c")
    base = wid * _B_PER_W
    idx_v = (idx0, idx1)
    rows_v = (rows0, rows1)
    sems = (sem0, sem1)
    pltpu.sync_copy(idx_hbm.at[pl.ds(base, _CHUNK)], idx0)
    cps = {0: pltpu.async_copy(table_hbm.at[idx0], rows0, sem0)}
    for ci in range(_N_CHUNKS):
        cur, nxt = ci % 2, (ci + 1) % 2
        if ci + 1 < _N_CHUNKS:
            off = base + (ci + 1) * _CHUNK
            pltpu.sync_copy(idx_hbm.at[pl.ds(off, _CHUNK)], idx_v[nxt])
            cps[nxt] = pltpu.async_copy(
                table_hbm.at[idx_v[nxt]], rows_v[nxt], sems[nxt])
        cps[cur].wait()
        pltpu.sync_copy(rows_v[cur], out_hbm.at[pl.ds(base + ci * _CHUNK, _CHUNK)])


@functools.cache
def _sc_gather():
    return functools.partial(
        pl.kernel,
        mesh=plsc.VectorSubcoreMesh(core_axis_name="c", subcore_axis_name="s"),
        out_type=jax.ShapeDtypeStruct((_GATHER_B, D_IN), jnp.float32),
        scratch_types=[
            pltpu.VMEM((_CHUNK,), jnp.int32),
            pltpu.VMEM((_CHUNK,), jnp.int32),
            pltpu.VMEM((_CHUNK, D_IN), jnp.float32),
            pltpu.VMEM((_CHUNK, D_IN), jnp.float32),
            pltpu.SemaphoreType.DMA,
            pltpu.SemaphoreType.DMA,
        ],
    )(_sc_gather_body)


def _mlp_body(g0_ref, g1_ref, g2_ref, w_ref, xs_ref,
              w1a_ref, w1b_ref, b1_ref, w2m_ref, b2_ref, o_ref):
    w = w_ref[...]
    xi = (w[:, 0:1] * g0_ref[...]
          + w[:, 1:2] * g1_ref[...]
          + w[:, 2:3] * g2_ref[...])
    h = jnp.dot(xi, w1a_ref[...], preferred_element_type=jnp.float32)
    h = h + jnp.dot(xs_ref[...], w1b_ref[...],
                    preferred_element_type=jnp.float32)
    h = jnp.maximum(h + b1_ref[...][None, :], 0.0)
    o = jnp.dot(h, w2m_ref[...], preferred_element_type=jnp.float32)
    o_ref[...] = jnp.maximum(o + b2_ref[...][None, :], 0.0)


def _mlp_call(gathered, w_all, x_skip, W1, b1, W2, b2):
    m = x_skip.shape[0]
    grid = (m // BMC,)
    nb = m // BMC
    return pl.pallas_call(
        _mlp_body,
        grid=grid,
        in_specs=[
            pl.BlockSpec((BMC, D_IN), lambda i: (i, 0)),
            pl.BlockSpec((BMC, D_IN), lambda i: (i + nb, 0)),
            pl.BlockSpec((BMC, D_IN), lambda i: (i + 2 * nb, 0)),
            pl.BlockSpec((BMC, 3), lambda i: (i, 0)),
            pl.BlockSpec((BMC, D_SKIP), lambda i: (i, 0)),
            pl.BlockSpec((D_IN, D_HID), lambda i: (0, 0)),
            pl.BlockSpec((D_SKIP, D_HID), lambda i: (2, 0)),
            pl.BlockSpec((D_HID,), lambda i: (0,)),
            pl.BlockSpec((D_HID, D_OUT), lambda i: (0, 0)),
            pl.BlockSpec((D_OUT,), lambda i: (0,)),
        ],
        out_specs=pl.BlockSpec((BMC, D_OUT), lambda i: (i, 0)),
        out_shape=jax.ShapeDtypeStruct((m, D_OUT), jnp.float32),
    )(gathered, gathered, gathered, w_all, x_skip, W1, W1, b1, W2, b2)


def kernel(x, pos, batch, x_skip, pos_skip, batch_skip, W1, b1, W2, b2):
    m = pos_skip.shape[0]
    idx_all, w_all = _topk_call(pos_skip, pos.T)

    gathered = _sc_gather()(x, idx_all.T.reshape(-1))
    h = _mlp_call(gathered, w_all, x_skip, W1, b1, W2, b2)
    return (h, pos_skip, batch_skip)


# speedup vs baseline: 1.3719x; 1.1020x over previous
import functools

import jax
import jax.numpy as jnp
import numpy as np
from jax import lax
from jax.experimental import pallas as pl
from jax.experimental.pallas import tpu as pltpu
from jax.experimental.pallas import tpu_sc as plsc

N_COARSE = 4096
N_FINE = 16384
D_IN = 256
D_SKIP = 128
D_HID = 256
D_OUT = 256
K = 3

BM = 1024
BMC = 512


def _topk_body(a_ref, b_ref, ysq_ref, i_ref, w_ref):
    d = jnp.dot(a_ref[...], b_ref[...], preferred_element_type=jnp.float32)
    n = d.shape[1]
    idxrow = lax.broadcasted_iota(jnp.int32, d.shape, 1).astype(jnp.float32)
    big_f = jnp.float32(n)
    inf = jnp.float32(np.inf)

    mins, idxs = [], []
    for _ in range(K):
        mk = jnp.min(d, axis=1, keepdims=True)
        cand = jnp.where(d == mk, idxrow, big_f)
        ik = jnp.min(cand, axis=1, keepdims=True)
        d = jnp.where(cand == ik, inf, d)
        mins.append(mk)
        idxs.append(ik)

    ysq = ysq_ref[...]
    ws = [1.0 / jnp.maximum(mk + ysq, 1e-16) for mk in mins]
    den = ws[0] + ws[1] + ws[2]
    i_ref[...] = jnp.concatenate(
        [ik.astype(jnp.int32) for ik in idxs], axis=1)
    w_ref[...] = jnp.concatenate([wk / den for wk in ws], axis=1)


_KSPLIT = 21


def _split3(v):
    p1 = v.astype(jnp.bfloat16)
    r = v - p1.astype(jnp.float32)
    p2 = r.astype(jnp.bfloat16)
    p3 = (r - p2.astype(jnp.float32)).astype(jnp.bfloat16)
    return p1, p2, p3


def _topk_operands(pos_skip, pos):
    u1, u2, u3 = _split3(pos_skip)
    v1, v2, v3 = _split3(-2.0 * pos.T)
    s1, s2, s3 = _split3(jnp.sum(pos * pos, axis=1)[None, :])
    one = jnp.ones(pos_skip.shape, jnp.bfloat16)
    a = jnp.concatenate([u1, u1, u2, u2, u1, u3, one], axis=1)
    b = jnp.concatenate(
        [v1, v2, v1, v2, v3, v1,
         jnp.concatenate([s1, s2, s3], axis=0)], axis=0)
    ysq = jnp.sum(pos_skip * pos_skip, axis=1, keepdims=True)
    return a, b, ysq


def _topk_call(a, b, ysq):
    m = a.shape[0]
    grid = (m // BM,)
    tri = pl.BlockSpec((BM, 3), lambda i: (i, 0))
    return pl.pallas_call(
        _topk_body,
        grid=grid,
        in_specs=[
            pl.BlockSpec((BM, _KSPLIT), lambda i: (i, 0)),
            pl.BlockSpec((_KSPLIT, N_COARSE), lambda i: (0, 0)),
            pl.BlockSpec((BM, 1), lambda i: (i, 0)),
        ],
        out_specs=[tri, tri],
        out_shape=[jax.ShapeDtypeStruct((m, 3), jnp.int32),
                   jax.ShapeDtypeStruct((m, 3), jnp.float32)],
    )(a, b, ysq)


_NC = 2
_NS = 16
_NW = _NC * _NS
_GATHER_B = K * N_FINE
_B_PER_W = _GATHER_B // _NW
_CHUNK = 192
_N_CHUNKS = _B_PER_W // _CHUNK


def _sc_gather_body(table_hbm, idx_hbm, out_hbm,
                    idx0, idx1, rows0, rows1, sem0, sem1):
    wid = lax.axis_index("s") * _NC + lax.axis_index("c")
    base = wid * _B_PER_W
    idx_v = (idx0, idx1)
    rows_v = (rows0, rows1)
    sems = (sem0, sem1)
    pltpu.sync_copy(idx_hbm.at[pl.ds(base, _CHUNK)], idx0)
    cps = {0: pltpu.async_copy(table_hbm.at[idx0], rows0, sem0)}
    for ci in range(_N_CHUNKS):
        cur, nxt = ci % 2, (ci + 1) % 2
        if ci + 1 < _N_CHUNKS:
            off = base + (ci + 1) * _CHUNK
            pltpu.sync_copy(idx_hbm.at[pl.ds(off, _CHUNK)], idx_v[nxt])
            cps[nxt] = pltpu.async_copy(
                table_hbm.at[idx_v[nxt]], rows_v[nxt], sems[nxt])
        cps[cur].wait()
        pltpu.sync_copy(rows_v[cur], out_hbm.at[pl.ds(base + ci * _CHUNK, _CHUNK)])


@functools.cache
def _sc_gather():
    return functools.partial(
        pl.kernel,
        mesh=plsc.VectorSubcoreMesh(core_axis_name="c", subcore_axis_name="s"),
        out_type=jax.ShapeDtypeStruct((_GATHER_B, D_IN), jnp.float32),
        scratch_types=[
            pltpu.VMEM((_CHUNK,), jnp.int32),
            pltpu.VMEM((_CHUNK,), jnp.int32),
            pltpu.VMEM((_CHUNK, D_IN), jnp.float32),
            pltpu.VMEM((_CHUNK, D_IN), jnp.float32),
            pltpu.SemaphoreType.DMA,
            pltpu.SemaphoreType.DMA,
        ],
    )(_sc_gather_body)


def _mlp_body(g0_ref, g1_ref, g2_ref, w_ref, xs_ref,
              w1a_ref, w1b_ref, b1_ref, w2m_ref, b2_ref, o_ref):
    w = w_ref[...]
    xi = (w[:, 0:1] * g0_ref[...]
          + w[:, 1:2] * g1_ref[...]
          + w[:, 2:3] * g2_ref[...])
    h = jnp.dot(xi, w1a_ref[...], preferred_element_type=jnp.float32)
    h = h + jnp.dot(xs_ref[...], w1b_ref[...],
                    preferred_element_type=jnp.float32)
    h = jnp.maximum(h + b1_ref[...][None, :], 0.0)
    o = jnp.dot(h, w2m_ref[...], preferred_element_type=jnp.float32)
    o_ref[...] = jnp.maximum(o + b2_ref[...][None, :], 0.0)


def _mlp_call(gathered, w_all, x_skip, W1, b1, W2, b2):
    m = x_skip.shape[0]
    grid = (m // BMC,)
    nb = m // BMC
    return pl.pallas_call(
        _mlp_body,
        grid=grid,
        in_specs=[
            pl.BlockSpec((BMC, D_IN), lambda i: (i, 0)),
            pl.BlockSpec((BMC, D_IN), lambda i: (i + nb, 0)),
            pl.BlockSpec((BMC, D_IN), lambda i: (i + 2 * nb, 0)),
            pl.BlockSpec((BMC, 3), lambda i: (i, 0)),
            pl.BlockSpec((BMC, D_SKIP), lambda i: (i, 0)),
            pl.BlockSpec((D_IN, D_HID), lambda i: (0, 0)),
            pl.BlockSpec((D_SKIP, D_HID), lambda i: (2, 0)),
            pl.BlockSpec((D_HID,), lambda i: (0,)),
            pl.BlockSpec((D_HID, D_OUT), lambda i: (0, 0)),
            pl.BlockSpec((D_OUT,), lambda i: (0,)),
        ],
        out_specs=pl.BlockSpec((BMC, D_OUT), lambda i: (i, 0)),
        out_shape=jax.ShapeDtypeStruct((m, D_OUT), jnp.float32),
    )(gathered, gathered, gathered, w_all, x_skip, W1, W1, b1, W2, b2)


def kernel(x, pos, batch, x_skip, pos_skip, batch_skip, W1, b1, W2, b2):
    m = pos_skip.shape[0]
    idx_all, w_all = _topk_call(*_topk_operands(pos_skip, pos))

    gathered = _sc_gather()(x, idx_all.T.reshape(-1))
    h = _mlp_call(gathered, w_all, x_skip, W1, b1, W2, b2)
    return (h, pos_skip, batch_skip)


# speedup vs baseline: 1.3856x; 1.0100x over previous
import functools

import jax
import jax.numpy as jnp
import numpy as np
from jax import lax
from jax.experimental import pallas as pl
from jax.experimental.pallas import tpu as pltpu
from jax.experimental.pallas import tpu_sc as plsc

N_COARSE = 4096
N_FINE = 16384
D_IN = 256
D_SKIP = 128
D_HID = 256
D_OUT = 256
K = 3

BM = 1024
BMC = 512


def _topk_body(a_ref, b_ref, ysq_ref, i_ref, w_ref):
    d = jnp.dot(a_ref[...], b_ref[...], preferred_element_type=jnp.float32)
    n = d.shape[1]
    idxrow = lax.broadcasted_iota(jnp.int32, d.shape, 1).astype(jnp.float32)
    big_f = jnp.float32(n)
    inf = jnp.float32(np.inf)

    mins, idxs = [], []
    for _ in range(K):
        mk = jnp.min(d, axis=1, keepdims=True)
        cand = jnp.where(d == mk, idxrow, big_f)
        ik = jnp.min(cand, axis=1, keepdims=True)
        d = jnp.where(cand == ik, inf, d)
        mins.append(mk)
        idxs.append(ik)

    ysq = ysq_ref[...]
    ws = [1.0 / jnp.maximum(mk + ysq, 1e-16) for mk in mins]
    den = ws[0] + ws[1] + ws[2]
    i_ref[...] = jnp.concatenate(
        [ik.astype(jnp.int32) for ik in idxs], axis=1)
    w_ref[...] = jnp.concatenate([wk / den for wk in ws], axis=1)


_KSPLIT = 21


def _hi16(v):
    return lax.bitcast_convert_type(
        lax.bitcast_convert_type(v, jnp.int32) & jnp.int32(-65536),
        jnp.float32)


def _split3(v):
    p1 = _hi16(v)
    r = v - p1
    p2 = _hi16(r)
    p3 = r - p2
    return (p1.astype(jnp.bfloat16), p2.astype(jnp.bfloat16),
            p3.astype(jnp.bfloat16))


def _topk_operands(pos_skip, pos):
    u1, u2, u3 = _split3(pos_skip)
    v1, v2, v3 = _split3(-2.0 * pos.T)
    s1, s2, s3 = _split3(jnp.sum(pos * pos, axis=1)[None, :])
    one = jnp.ones(pos_skip.shape, jnp.bfloat16)
    a = jnp.concatenate([u1, u1, u2, u2, u1, u3, one], axis=1)
    b = jnp.concatenate(
        [v1, v2, v1, v2, v3, v1,
         jnp.concatenate([s1, s2, s3], axis=0)], axis=0)
    ysq = jnp.sum(pos_skip * pos_skip, axis=1, keepdims=True)
    return a, b, ysq


def _topk_call(a, b, ysq):
    m = a.shape[0]
    grid = (m // BM,)
    tri = pl.BlockSpec((BM, 3), lambda i: (i, 0))
    return pl.pallas_call(
        _topk_body,
        grid=grid,
        in_specs=[
            pl.BlockSpec((BM, _KSPLIT), lambda i: (i, 0)),
            pl.BlockSpec((_KSPLIT, N_COARSE), lambda i: (0, 0)),
            pl.BlockSpec((BM, 1), lambda i: (i, 0)),
        ],
        out_specs=[tri, tri],
        out_shape=[jax.ShapeDtypeStruct((m, 3), jnp.int32),
                   jax.ShapeDtypeStruct((m, 3), jnp.float32)],
    )(a, b, ysq)


_NC = 2
_NS = 16
_NW = _NC * _NS
_GATHER_B = K * N_FINE
_B_PER_W = _GATHER_B // _NW
_CHUNK = 192
_N_CHUNKS = _B_PER_W // _CHUNK


def _sc_gather_body(table_hbm, idx_hbm, out_hbm,
                    idx0, idx1, rows0, rows1, sem0, sem1):
    wid = lax.axis_index("s") * _NC + lax.axis_index("c")
    base = wid * _B_PER_W
    idx_v = (idx0, idx1)
    rows_v = (rows0, rows1)
    sems = (sem0, sem1)
    pltpu.sync_copy(idx_hbm.at[pl.ds(base, _CHUNK)], idx0)
    cps = {0: pltpu.async_copy(table_hbm.at[idx0], rows0, sem0)}
    for ci in range(_N_CHUNKS):
        cur, nxt = ci % 2, (ci + 1) % 2
        if ci + 1 < _N_CHUNKS:
            off = base + (ci + 1) * _CHUNK
            pltpu.sync_copy(idx_hbm.at[pl.ds(off, _CHUNK)], idx_v[nxt])
            cps[nxt] = pltpu.async_copy(
                table_hbm.at[idx_v[nxt]], rows_v[nxt], sems[nxt])
        cps[cur].wait()
        pltpu.sync_copy(rows_v[cur], out_hbm.at[pl.ds(base + ci * _CHUNK, _CHUNK)])


@functools.cache
def _sc_gather():
    return functools.partial(
        pl.kernel,
        mesh=plsc.VectorSubcoreMesh(core_axis_name="c", subcore_axis_name="s"),
        out_type=jax.ShapeDtypeStruct((_GATHER_B, D_IN), jnp.float32),
        scratch_types=[
            pltpu.VMEM((_CHUNK,), jnp.int32),
            pltpu.VMEM((_CHUNK,), jnp.int32),
            pltpu.VMEM((_CHUNK, D_IN), jnp.float32),
            pltpu.VMEM((_CHUNK, D_IN), jnp.float32),
            pltpu.SemaphoreType.DMA,
            pltpu.SemaphoreType.DMA,
        ],
    )(_sc_gather_body)


def _mlp_body(g0_ref, g1_ref, g2_ref, w_ref, xs_ref,
              w1a_ref, w1b_ref, b1_ref, w2m_ref, b2_ref, o_ref):
    w = w_ref[...]
    xi = (w[:, 0:1] * g0_ref[...]
          + w[:, 1:2] * g1_ref[...]
          + w[:, 2:3] * g2_ref[...])
    h = jnp.dot(xi, w1a_ref[...], preferred_element_type=jnp.float32)
    h = h + jnp.dot(xs_ref[...], w1b_ref[...],
                    preferred_element_type=jnp.float32)
    h = jnp.maximum(h + b1_ref[...][None, :], 0.0)
    o = jnp.dot(h, w2m_ref[...], preferred_element_type=jnp.float32)
    o_ref[...] = jnp.maximum(o + b2_ref[...][None, :], 0.0)


def _mlp_call(gathered, w_all, x_skip, W1, b1, W2, b2):
    m = x_skip.shape[0]
    grid = (m // BMC,)
    nb = m // BMC
    return pl.pallas_call(
        _mlp_body,
        grid=grid,
        in_specs=[
            pl.BlockSpec((BMC, D_IN), lambda i: (i, 0)),
            pl.BlockSpec((BMC, D_IN), lambda i: (i + nb, 0)),
            pl.BlockSpec((BMC, D_IN), lambda i: (i + 2 * nb, 0)),
            pl.BlockSpec((BMC, 3), lambda i: (i, 0)),
            pl.BlockSpec((BMC, D_SKIP), lambda i: (i, 0)),
            pl.BlockSpec((D_IN, D_HID), lambda i: (0, 0)),
            pl.BlockSpec((D_SKIP, D_HID), lambda i: (2, 0)),
            pl.BlockSpec((D_HID,), lambda i: (0,)),
            pl.BlockSpec((D_HID, D_OUT), lambda i: (0, 0)),
            pl.BlockSpec((D_OUT,), lambda i: (0,)),
        ],
        out_specs=pl.BlockSpec((BMC, D_OUT), lambda i: (i, 0)),
        out_shape=jax.ShapeDtypeStruct((m, D_OUT), jnp.float32),
    )(gathered, gathered, gathered, w_all, x_skip, W1, W1, b1, W2, b2)


def kernel(x, pos, batch, x_skip, pos_skip, batch_skip, W1, b1, W2, b2):
    m = pos_skip.shape[0]
    idx_all, w_all = _topk_call(*_topk_operands(pos_skip, pos))

    gathered = _sc_gather()(x, idx_all.T.reshape(-1))
    h = _mlp_call(gathered, w_all, x_skip, W1, b1, W2, b2)
    return (h, pos_skip, batch_skip)


# speedup vs baseline: 1.4397x; 1.0391x over previous
import functools

import jax
import jax.numpy as jnp
import numpy as np
from jax import lax
from jax.experimental import pallas as pl
from jax.experimental.pallas import tpu as pltpu
from jax.experimental.pallas import tpu_sc as plsc

N_COARSE = 4096
N_FINE = 16384
D_IN = 256
D_SKIP = 128
D_HID = 256
D_OUT = 256
K = 3

BM = 1024
BMC = 1024


def _topk_body(a_ref, b_ref, ysq_ref, i_ref, w_ref):
    d = jnp.dot(a_ref[...], b_ref[...], preferred_element_type=jnp.float32)
    n = d.shape[1]
    idxrow = lax.broadcasted_iota(jnp.int32, d.shape, 1).astype(jnp.float32)
    big_f = jnp.float32(n)
    inf = jnp.float32(np.inf)

    mins, idxs = [], []
    for _ in range(K):
        mk = jnp.min(d, axis=1, keepdims=True)
        cand = jnp.where(d == mk, idxrow, big_f)
        ik = jnp.min(cand, axis=1, keepdims=True)
        d = jnp.where(cand == ik, inf, d)
        mins.append(mk)
        idxs.append(ik)

    ysq = ysq_ref[...]
    ws = [1.0 / jnp.maximum(mk + ysq, 1e-16) for mk in mins]
    den = ws[0] + ws[1] + ws[2]
    i_ref[...] = jnp.concatenate(
        [ik.astype(jnp.int32) for ik in idxs], axis=1)
    w_ref[...] = jnp.concatenate([wk / den for wk in ws], axis=1)


_KSPLIT = 21


def _hi16(v):
    return lax.bitcast_convert_type(
        lax.bitcast_convert_type(v, jnp.int32) & jnp.int32(-65536),
        jnp.float32)


def _split3(v):
    p1 = _hi16(v)
    r = v - p1
    p2 = _hi16(r)
    p3 = r - p2
    return (p1.astype(jnp.bfloat16), p2.astype(jnp.bfloat16),
            p3.astype(jnp.bfloat16))


def _topk_operands(pos_skip, pos):
    u1, u2, u3 = _split3(pos_skip)
    v1, v2, v3 = _split3(-2.0 * pos.T)
    s1, s2, s3 = _split3(jnp.sum(pos * pos, axis=1)[None, :])
    one = jnp.ones(pos_skip.shape, jnp.bfloat16)
    a = jnp.concatenate([u1, u1, u2, u2, u1, u3, one], axis=1)
    b = jnp.concatenate(
        [v1, v2, v1, v2, v3, v1,
         jnp.concatenate([s1, s2, s3], axis=0)], axis=0)
    ysq = jnp.sum(pos_skip * pos_skip, axis=1, keepdims=True)
    return a, b, ysq


def _topk_call(a, b, ysq):
    m = a.shape[0]
    grid = (m // BM,)
    tri = pl.BlockSpec((BM, 3), lambda i: (i, 0))
    return pl.pallas_call(
        _topk_body,
        grid=grid,
        in_specs=[
            pl.BlockSpec((BM, _KSPLIT), lambda i: (i, 0)),
            pl.BlockSpec((_KSPLIT, N_COARSE), lambda i: (0, 0)),
            pl.BlockSpec((BM, 1), lambda i: (i, 0)),
        ],
        out_specs=[tri, tri],
        out_shape=[jax.ShapeDtypeStruct((m, 3), jnp.int32),
                   jax.ShapeDtypeStruct((m, 3), jnp.float32)],
    )(a, b, ysq)


_NC = 2
_NS = 16
_NW = _NC * _NS
_GATHER_B = K * N_FINE
_B_PER_W = _GATHER_B // _NW
_CHUNK = 192
_N_CHUNKS = _B_PER_W // _CHUNK


def _sc_gather_body(table_hbm, idx_hbm, out_hbm,
                    idx0, idx1, rows0, rows1, sem0, sem1):
    wid = lax.axis_index("s") * _NC + lax.axis_index("c")
    base = wid * _B_PER_W
    idx_v = (idx0, idx1)
    rows_v = (rows0, rows1)
    sems = (sem0, sem1)
    pltpu.sync_copy(idx_hbm.at[pl.ds(base, _CHUNK)], idx0)
    cps = {0: pltpu.async_copy(table_hbm.at[idx0], rows0, sem0)}
    for ci in range(_N_CHUNKS):
        cur, nxt = ci % 2, (ci + 1) % 2
        if ci + 1 < _N_CHUNKS:
            off = base + (ci + 1) * _CHUNK
            pltpu.sync_copy(idx_hbm.at[pl.ds(off, _CHUNK)], idx_v[nxt])
            cps[nxt] = pltpu.async_copy(
                table_hbm.at[idx_v[nxt]], rows_v[nxt], sems[nxt])
        cps[cur].wait()
        pltpu.sync_copy(rows_v[cur], out_hbm.at[pl.ds(base + ci * _CHUNK, _CHUNK)])


@functools.cache
def _sc_gather():
    return functools.partial(
        pl.kernel,
        mesh=plsc.VectorSubcoreMesh(core_axis_name="c", subcore_axis_name="s"),
        out_type=jax.ShapeDtypeStruct((_GATHER_B, D_IN), jnp.float32),
        scratch_types=[
            pltpu.VMEM((_CHUNK,), jnp.int32),
            pltpu.VMEM((_CHUNK,), jnp.int32),
            pltpu.VMEM((_CHUNK, D_IN), jnp.float32),
            pltpu.VMEM((_CHUNK, D_IN), jnp.float32),
            pltpu.SemaphoreType.DMA,
            pltpu.SemaphoreType.DMA,
        ],
    )(_sc_gather_body)


def _mlp_body(g0_ref, g1_ref, g2_ref, w_ref, xs_ref,
              w1a_ref, w1b_ref, b1_ref, w2m_ref, b2_ref, o_ref):
    w = w_ref[...]
    xi = (w[:, 0:1] * g0_ref[...]
          + w[:, 1:2] * g1_ref[...]
          + w[:, 2:3] * g2_ref[...])
    h = jnp.dot(xi, w1a_ref[...], preferred_element_type=jnp.float32)
    h = h + jnp.dot(xs_ref[...], w1b_ref[...],
                    preferred_element_type=jnp.float32)
    h = jnp.maximum(h + b1_ref[...][None, :], 0.0)
    o = jnp.dot(h, w2m_ref[...], preferred_element_type=jnp.float32)
    o_ref[...] = jnp.maximum(o + b2_ref[...][None, :], 0.0)


def _mlp_call(gathered, w_all, x_skip, W1, b1, W2, b2):
    m = x_skip.shape[0]
    grid = (m // BMC,)
    nb = m // BMC
    return pl.pallas_call(
        _mlp_body,
        grid=grid,
        in_specs=[
            pl.BlockSpec((BMC, D_IN), lambda i: (i, 0)),
            pl.BlockSpec((BMC, D_IN), lambda i: (i + nb, 0)),
            pl.BlockSpec((BMC, D_IN), lambda i: (i + 2 * nb, 0)),
            pl.BlockSpec((BMC, 3), lambda i: (i, 0)),
            pl.BlockSpec((BMC, D_SKIP), lambda i: (i, 0)),
            pl.BlockSpec((D_IN, D_HID), lambda i: (0, 0)),
            pl.BlockSpec((D_SKIP, D_HID), lambda i: (2, 0)),
            pl.BlockSpec((D_HID,), lambda i: (0,)),
            pl.BlockSpec((D_HID, D_OUT), lambda i: (0, 0)),
            pl.BlockSpec((D_OUT,), lambda i: (0,)),
        ],
        out_specs=pl.BlockSpec((BMC, D_OUT), lambda i: (i, 0)),
        out_shape=jax.ShapeDtypeStruct((m, D_OUT), jnp.float32),
    )(gathered, gathered, gathered, w_all, x_skip, W1, W1, b1, W2, b2)


def kernel(x, pos, batch, x_skip, pos_skip, batch_skip, W1, b1, W2, b2):
    m = pos_skip.shape[0]
    idx_all, w_all = _topk_call(*_topk_operands(pos_skip, pos))

    gathered = _sc_gather()(x, idx_all.T.reshape(-1))
    h = _mlp_call(gathered, w_all, x_skip, W1, b1, W2, b2)
    return (h, pos_skip, batch_skip)
